# Initial kernel scaffold; baseline (speedup 1.0000x reference)
import jax
import jax.numpy as jnp
from jax.experimental import pallas as pl


def kernel(f1, f2, f3, f4, p_3_1_dw, p_3_1_pw, p_3_1_gamma, p_3_1_beta, p_3_1_mean, p_3_1_var, p_2_1_dw, p_2_1_pw, p_2_1_gamma, p_2_1_beta, p_2_1_mean, p_2_1_var, p_1_1_dw, p_1_1_pw, p_1_1_gamma, p_1_1_beta, p_1_1_mean, p_1_1_var, p_2_2_dw, p_2_2_pw, p_2_2_gamma, p_2_2_beta, p_2_2_mean, p_2_2_var, p_3_2_dw, p_3_2_pw, p_3_2_gamma, p_3_2_beta, p_3_2_mean, p_3_2_var, p_4_2_dw, p_4_2_pw, p_4_2_gamma, p_4_2_beta, p_4_2_mean, p_4_2_var):
    raise NotImplementedError("write your pallas kernel here")


# speedup vs baseline: 2.0544x; 2.0544x over previous
import functools
import math

import jax
import jax.numpy as jnp
from jax.experimental import pallas as pl
from jax.experimental.pallas import tpu as pltpu

_BN_EPS = 1e-5


def _row_block(ho, wo):
    rb = min(ho, max(8, 1024 // wo))
    while ho % rb:
        rb //= 2
    return rb


def _upsample_add_to_scratch(s_ref, x, y_quad, hs, ws):
    h, w = 2 * hs, 2 * ws
    xu = jnp.concatenate([x[:1], x[:-1]], axis=0)
    xd = jnp.concatenate([x[1:], x[-1:]], axis=0)
    xe = 0.75 * x + 0.25 * xu
    xo = 0.75 * x + 0.25 * xd

    def wsplit(v):
        vl = jnp.concatenate([v[:, :1], v[:, :-1]], axis=1)
        vr = jnp.concatenate([v[:, 1:], v[:, -1:]], axis=1)
        return 0.75 * v + 0.25 * vl, 0.75 * v + 0.25 * vr

    ee, eo = wsplit(xe)
    oe, oo = wsplit(xo)
    c = s_ref.shape[-1]
    s_ref[pl.ds(0, 1), pl.ds(0, w + 2), :] = jnp.zeros((1, w + 2, c), jnp.float32)
    s_ref[pl.ds(h + 1, 1), pl.ds(0, w + 2), :] = jnp.zeros((1, w + 2, c), jnp.float32)
    s_ref[pl.ds(0, h + 2), pl.ds(0, 1), :] = jnp.zeros((h + 2, 1, c), jnp.float32)
    s_ref[pl.ds(0, h + 2), pl.ds(w + 1, 1), :] = jnp.zeros((h + 2, 1, c), jnp.float32)
    s_ref[pl.ds(1, hs, 2), pl.ds(1, ws, 2), :] = ee + y_quad(0, 0)
    s_ref[pl.ds(1, hs, 2), pl.ds(2, ws, 2), :] = eo + y_quad(0, 1)
    s_ref[pl.ds(2, hs, 2), pl.ds(1, ws, 2), :] = oe + y_quad(1, 0)
    s_ref[pl.ds(2, hs, 2), pl.ds(2, ws, 2), :] = oo + y_quad(1, 1)


def _dw_pw_from_scratch(s_ref, dw_ref, pw_ref, b_ref, stride, ho, wo, store):
    c = s_ref.shape[-1]
    rb = _row_block(ho, wo)
    dwv = [dw_ref[k] for k in range(9)]
    bias = b_ref[...]
    for base in range(0, ho, rb):
        acc = jnp.zeros((rb, wo, c), jnp.float32)
        for k in range(9):
            ki, kj = k // 3, k % 3
            if stride == 1:
                tap = s_ref[pl.ds(base + ki, rb), pl.ds(kj, wo), :]
            else:
                tap = s_ref[pl.ds(2 * base + ki, rb, 2),
                            pl.ds(kj, wo, 2), :]
            acc = acc + tap * dwv[k]
        y = jnp.dot(acc.reshape(rb * wo, c).astype(jnp.bfloat16),
                    pw_ref[...], preferred_element_type=jnp.float32)
        y = jnp.maximum(y + bias, 0.0)
        store(base, rb, y.reshape(rb, wo, c))


def _fpem_kernel(x1, x2, x3, x4,
                 dw31, pw31, b31, dw21, pw21, b21, dw11, pw11, b11,
                 dw22, pw22, b22, dw32, pw32, b32, dw42, pw42, b42,
                 o1, o2, o3, o4, s_ref, f2p, f3p):
    h1, w1 = x1.shape[1], x1.shape[2]
    h2, w2 = x2.shape[1], x2.shape[2]
    h3, w3 = x3.shape[1], x3.shape[2]
    h4, w4 = x4.shape[1], x4.shape[2]

    def quad4(ref):
        return lambda a, b, hs=None, ws=None: ref[
            0, pl.ds(a, hs, 2), pl.ds(b, ws, 2), :]

    def quad3(ref):
        return lambda a, b, hs=None, ws=None: ref[
            pl.ds(a, hs, 2), pl.ds(b, ws, 2), :]

    def store3(ref):
        return lambda base, rb, v: ref.__setitem__(
            (pl.ds(base, rb), slice(None), slice(None)), v)

    def store4(ref):
        return lambda base, rb, v: ref.__setitem__(
            (0, pl.ds(base, rb), slice(None), slice(None)), v)

    def level(x_val, hs, ws, yq, dw, pw, b, stride, store):
        yqf = lambda a, b_: yq(a, b_, hs=hs, ws=ws)
        _upsample_add_to_scratch(s_ref, x_val, yqf, hs, ws)
        if stride == 1:
            ho, wo = 2 * hs, 2 * ws
        else:
            ho, wo = hs, ws
        _dw_pw_from_scratch(s_ref, dw, pw, b, stride, ho, wo, store)

    level(x4[0], h4, w4, quad4(x3), dw31, pw31, b31, 1, store3(f3p))
    level(f3p[...], h3, w3, quad4(x2), dw21, pw21, b21, 1, store3(f2p))
    level(f2p[...], h2, w2, quad4(x1), dw11, pw11, b11, 1, store4(o1))
    level(f2p[...], h2, w2, quad4(o1), dw22, pw22, b22, 2, store4(o2))
    level(f3p[...], h3, w3, quad4(o2), dw32, pw32, b32, 2, store4(o3))
    level(x4[0], h4, w4, quad4(o3), dw42, pw42, b42, 2, store4(o4))


def _fold_params(dw, pw, gamma, beta, mean, var):
    scale = gamma * jax.lax.rsqrt(var + _BN_EPS)
    bias = (beta - mean * scale)[None, :]
    pwf = (pw * scale[None, :]).astype(jnp.bfloat16)
    c = dw.shape[-1]
    return dw.reshape(9, c), pwf, bias


@jax.jit
def kernel(f1, f2, f3, f4, p_3_1_dw, p_3_1_pw, p_3_1_gamma, p_3_1_beta, p_3_1_mean, p_3_1_var, p_2_1_dw, p_2_1_pw, p_2_1_gamma, p_2_1_beta, p_2_1_mean, p_2_1_var, p_1_1_dw, p_1_1_pw, p_1_1_gamma, p_1_1_beta, p_1_1_mean, p_1_1_var, p_2_2_dw, p_2_2_pw, p_2_2_gamma, p_2_2_beta, p_2_2_mean, p_2_2_var, p_3_2_dw, p_3_2_pw, p_3_2_gamma, p_3_2_beta, p_3_2_mean, p_3_2_var, p_4_2_dw, p_4_2_pw, p_4_2_gamma, p_4_2_beta, p_4_2_mean, p_4_2_var):
    to_nhwc = lambda t: jnp.transpose(t, (0, 2, 3, 1))
    x1, x2, x3, x4 = to_nhwc(f1), to_nhwc(f2), to_nhwc(f3), to_nhwc(f4)
    n, h1, w1, c = x1.shape
    h2, w2 = x2.shape[1], x2.shape[2]
    h3, w3 = x3.shape[1], x3.shape[2]
    h4, w4 = x4.shape[1], x4.shape[2]

    folded = []
    folded += _fold_params(p_3_1_dw, p_3_1_pw, p_3_1_gamma, p_3_1_beta,
                           p_3_1_mean, p_3_1_var)
    folded += _fold_params(p_2_1_dw, p_2_1_pw, p_2_1_gamma, p_2_1_beta,
                           p_2_1_mean, p_2_1_var)
    folded += _fold_params(p_1_1_dw, p_1_1_pw, p_1_1_gamma, p_1_1_beta,
                           p_1_1_mean, p_1_1_var)
    folded += _fold_params(p_2_2_dw, p_2_2_pw, p_2_2_gamma, p_2_2_beta,
                           p_2_2_mean, p_2_2_var)
    folded += _fold_params(p_3_2_dw, p_3_2_pw, p_3_2_gamma, p_3_2_beta,
                           p_3_2_mean, p_3_2_var)
    folded += _fold_params(p_4_2_dw, p_4_2_pw, p_4_2_gamma, p_4_2_beta,
                           p_4_2_mean, p_4_2_var)

    fmap = lambda i: (i, 0, 0, 0)
    x_specs = [
        pl.BlockSpec((1, h1, w1, c), fmap),
        pl.BlockSpec((1, h2, w2, c), fmap),
        pl.BlockSpec((1, h3, w3, c), fmap),
        pl.BlockSpec((1, h4, w4, c), fmap),
    ]
    w_specs = []
    for _ in range(6):
        w_specs += [
            pl.BlockSpec((9, c), lambda i: (0, 0)),
            pl.BlockSpec((c, c), lambda i: (0, 0)),
            pl.BlockSpec((1, c), lambda i: (0, 0)),
        ]

    m_total = n * (h1 * w1 + 2 * h2 * w2 + 2 * h3 * w3 + h4 * w4)
    flops = 2 * m_total * c * c + 2 * 9 * m_total * c
    bytes_accessed = 8 * n * (h1 * w1 + h2 * w2 + h3 * w3 + h4 * w4) * c

    out = pl.pallas_call(
        _fpem_kernel,
        grid=(n,),
        in_specs=x_specs + w_specs,
        out_specs=[
            pl.BlockSpec((1, h1, w1, c), fmap),
            pl.BlockSpec((1, h2, w2, c), fmap),
            pl.BlockSpec((1, h3, w3, c), fmap),
            pl.BlockSpec((1, h4, w4, c), fmap),
        ],
        out_shape=[
            jax.ShapeDtypeStruct((n, h1, w1, c), jnp.float32),
            jax.ShapeDtypeStruct((n, h2, w2, c), jnp.float32),
            jax.ShapeDtypeStruct((n, h3, w3, c), jnp.float32),
            jax.ShapeDtypeStruct((n, h4, w4, c), jnp.float32),
        ],
        scratch_shapes=[
            pltpu.VMEM((h1 + 2, w1 + 2, c), jnp.float32),
            pltpu.VMEM((h2, w2, c), jnp.float32),
            pltpu.VMEM((h3, w3, c), jnp.float32),
        ],
        compiler_params=pltpu.CompilerParams(
            dimension_semantics=("parallel",),
            vmem_limit_bytes=100 * 1024 * 1024,
        ),
        cost_estimate=pl.CostEstimate(flops=int(flops), transcendentals=0,
                                      bytes_accessed=int(bytes_accessed)),
    )(x1, x2, x3, x4, *folded)

    to_nchw = lambda t: jnp.transpose(t, (0, 3, 1, 2))
    return tuple(to_nchw(t) for t in out)
